# baseline (speedup 1.0000x reference)
import functools

import jax
import jax.numpy as jnp
from jax import lax
from jax.experimental import pallas as pl
from jax.experimental.pallas import tpu as pltpu
from jax.experimental.pallas import tpu_sc as plsc

N = 10000
D_IN = 128
DH = 16
DOUT = 128
NW = 32
NB = 80
CB = 128
EPT = NB * CB
E_PAD = NW * EPT
NBUF = 8
RPT = 624
RPT_LAST = N - 15 * RPT
BLK = 1000


@functools.partial(
    pl.kernel,
    mesh=plsc.VectorSubcoreMesh(core_axis_name="c", subcore_axis_name="s"),
    out_type=jax.ShapeDtypeStruct((2, N, DH), jnp.float32),
    scratch_types=[
        pltpu.VMEM((NB, CB), jnp.int32),
        pltpu.VMEM((NB, CB), jnp.int32),
        pltpu.VMEM((NB, CB), jnp.float32),
        pltpu.VMEM((NBUF, CB, DH), jnp.float32),
        pltpu.VMEM((NBUF, CB, DH), jnp.float32),
        pltpu.VMEM((RPT_LAST, DH), jnp.float32),
        pltpu.VMEM_SHARED((N, DH), jnp.float32),
        pltpu.SemaphoreType.DMA((3,)),
        pltpu.SemaphoreType.DMA((NBUF,)),
        pltpu.SemaphoreType.DMA((NBUF,)),
    ],
    compiler_params=pltpu.CompilerParams(use_tc_tiling_on_sc=False),
)
def _sc_edge_agg(table, src3, dst3, w3, out, src_v, dst_v, w_v, g_rows,
                 s_rows, zbuf, acc, e_sem, g_sem, s_sem):
    cid = lax.axis_index("c")
    sid = lax.axis_index("s")
    wid = sid * 2 + cid

    cp_src = pltpu.async_copy(src3.at[wid], src_v, e_sem.at[0])
    cp_dst = pltpu.async_copy(dst3.at[wid], dst_v, e_sem.at[1])
    cp_w = pltpu.async_copy(w3.at[wid], w_v, e_sem.at[2])

    def _zrow(i, carry):
        zbuf[i] = jnp.zeros((DH,), jnp.float32)
        return carry

    lax.fori_loop(0, RPT_LAST, _zrow, 0)
    cp_src.wait()
    cp_dst.wait()
    cp_w.wait()

    @pl.when(sid < 15)
    def _():
        pltpu.sync_copy(zbuf.at[pl.ds(0, RPT)], acc.at[pl.ds(sid * RPT, RPT)])

    @pl.when(sid == 15)
    def _():
        pltpu.sync_copy(zbuf, acc.at[pl.ds(15 * RPT, RPT_LAST)])

    plsc.subcore_barrier()

    for k in range(NBUF):
        pltpu.async_copy(table.at[src_v.at[k]], g_rows.at[k], g_sem.at[k])

    def _group(g, carry):
        for k in range(NBUF):
            b = g * NBUF + k
            pltpu.make_async_copy(table.at[src_v.at[b]], g_rows.at[k],
                                  g_sem.at[k]).wait()

            @pl.when(g > 0)
            def _():
                pltpu.make_async_copy(s_rows.at[k], acc.at[dst_v.at[b - NBUF]],
                                      s_sem.at[k]).wait()

            for gg in range(CB // 16):
                wv = w_v[b, pl.ds(gg * 16, 16)]
                for l in range(16):
                    i = gg * 16 + l
                    s_rows[k, i] = g_rows[k, i] * wv[l]

            @pl.when(g < NB // NBUF - 1)
            def _():
                pltpu.async_copy(table.at[src_v.at[b + NBUF]], g_rows.at[k],
                                 g_sem.at[k])

            pltpu.async_copy(s_rows.at[k], acc.at[dst_v.at[b]], s_sem.at[k],
                             add=True)
        return carry

    lax.fori_loop(0, NB // NBUF, _group, 0)
    for k in range(NBUF):
        pltpu.make_async_copy(s_rows.at[k], acc.at[dst_v.at[NB - NBUF + k]],
                              s_sem.at[k]).wait()

    plsc.subcore_barrier()

    @pl.when(sid < 15)
    def _():
        pltpu.sync_copy(acc.at[pl.ds(sid * RPT, RPT)],
                        out.at[cid, pl.ds(sid * RPT, RPT)])

    @pl.when(sid == 15)
    def _():
        pltpu.sync_copy(acc.at[pl.ds(15 * RPT, RPT_LAST)],
                        out.at[cid, pl.ds(15 * RPT, RPT_LAST)])


def _lin1_body(x_ref, w_ref, xr_ref, xo_ref):
    acc = jnp.dot(x_ref[...], w_ref[...], preferred_element_type=jnp.float32)
    xr_ref[...] = acc[:, :DH]
    xo_ref[...] = acc[:, DH:]


def _lin1(x, wcat):
    return pl.pallas_call(
        _lin1_body,
        grid=(N // BLK,),
        in_specs=[pl.BlockSpec((BLK, D_IN), lambda i: (i, 0)),
                  pl.BlockSpec((D_IN, 2 * DH), lambda i: (0, 0))],
        out_specs=[pl.BlockSpec((BLK, DH), lambda i: (i, 0)),
                   pl.BlockSpec((BLK, DH), lambda i: (i, 0))],
        out_shape=[jax.ShapeDtypeStruct((N, DH), jnp.float32),
                   jax.ShapeDtypeStruct((N, DH), jnp.float32)],
    )(x, wcat)


def _hidden_body(p_ref, xo_ref, b_ref, h_ref):
    h_ref[...] = jnp.maximum(p_ref[0] + p_ref[1] + xo_ref[...] + b_ref[...],
                             0.0)


def _hidden(p, xo, b1):
    return pl.pallas_call(
        _hidden_body,
        grid=(N // BLK,),
        in_specs=[pl.BlockSpec((2, BLK, DH), lambda i: (0, i, 0)),
                  pl.BlockSpec((BLK, DH), lambda i: (i, 0)),
                  pl.BlockSpec((1, DH), lambda i: (0, 0))],
        out_specs=pl.BlockSpec((BLK, DH), lambda i: (i, 0)),
        out_shape=jax.ShapeDtypeStruct((N, DH), jnp.float32),
    )(p, xo, b1)


def _out_body(q_ref, h_ref, wr_ref, wo_ref, b_ref, o_ref):
    agg = q_ref[0] + q_ref[1]
    o_ref[...] = (jnp.dot(agg, wr_ref[...], preferred_element_type=jnp.float32)
                  + jnp.dot(h_ref[...], wo_ref[...],
                            preferred_element_type=jnp.float32)
                  + b_ref[...])


def _out(q, h, wr, wo, b2):
    return pl.pallas_call(
        _out_body,
        grid=(N // BLK,),
        in_specs=[pl.BlockSpec((2, BLK, DH), lambda i: (0, i, 0)),
                  pl.BlockSpec((BLK, DH), lambda i: (i, 0)),
                  pl.BlockSpec((DH, DOUT), lambda i: (0, 0)),
                  pl.BlockSpec((DH, DOUT), lambda i: (0, 0)),
                  pl.BlockSpec((1, DOUT), lambda i: (0, 0))],
        out_specs=pl.BlockSpec((BLK, DOUT), lambda i: (i, 0)),
        out_shape=jax.ShapeDtypeStruct((N, DOUT), jnp.float32),
    )(q, h, wr, wo, b2)


def kernel(x, edge_index, edge_attr, W_rel1, b_rel1, W_root1, W_rel2, b_rel2,
           W_root2):
    e = edge_attr.shape[0]
    pad = E_PAD - e
    src3 = jnp.pad(edge_index[0], (0, pad)).reshape(NW, NB, CB)
    dst3 = jnp.pad(edge_index[1], (0, pad)).reshape(NW, NB, CB)
    w3 = jnp.pad(edge_attr, (0, pad)).reshape(NW, NB, CB)

    xr, xo = _lin1(x, jnp.concatenate([W_rel1, W_root1], axis=1))
    p = _sc_edge_agg(xr, src3, dst3, w3)
    h = _hidden(p, xo, b_rel1.reshape(1, DH))
    q = _sc_edge_agg(h, src3, dst3, w3)
    return _out(q, h, W_rel2, W_root2, b_rel2.reshape(1, DOUT))


# speedup vs baseline: 1.1755x; 1.1755x over previous
import functools

import jax
import jax.numpy as jnp
from jax import lax
from jax.experimental import pallas as pl
from jax.experimental.pallas import tpu as pltpu
from jax.experimental.pallas import tpu_sc as plsc

N = 10000
D_IN = 128
DH = 16
DOUT = 128
NW = 32
NB = 80
CB = 128
EPT = NB * CB
E_PAD = NW * EPT
NBUF = 8
RPT = 624
RPT_LAST = N - 15 * RPT
BLK = 1000


@functools.partial(
    pl.kernel,
    mesh=plsc.VectorSubcoreMesh(core_axis_name="c", subcore_axis_name="s"),
    out_type=jax.ShapeDtypeStruct((2, N, DH), jnp.float32),
    scratch_types=[
        pltpu.VMEM((NB, CB), jnp.int32),
        pltpu.VMEM((NB, CB), jnp.int32),
        pltpu.VMEM((NB, CB), jnp.float32),
        pltpu.VMEM((NBUF, CB, DH), jnp.float32),
        pltpu.VMEM((NBUF, CB, DH), jnp.float32),
        pltpu.VMEM((RPT_LAST, DH), jnp.float32),
        pltpu.VMEM_SHARED((N, DH), jnp.float32),
        pltpu.SemaphoreType.DMA((3,)),
        pltpu.SemaphoreType.DMA((NBUF,)),
        pltpu.SemaphoreType.DMA((NBUF,)),
    ],
    compiler_params=pltpu.CompilerParams(use_tc_tiling_on_sc=False),
)
def _sc_edge_agg(table, src3, dst3, w3, out, src_v, dst_v, w_v, g_rows,
                 s_rows, zbuf, acc, e_sem, g_sem, s_sem):
    cid = lax.axis_index("c")
    sid = lax.axis_index("s")
    wid = sid * 2 + cid

    cp_src = pltpu.async_copy(src3.at[wid], src_v, e_sem.at[0])
    cp_dst = pltpu.async_copy(dst3.at[wid], dst_v, e_sem.at[1])
    cp_w = pltpu.async_copy(w3.at[wid], w_v, e_sem.at[2])

    def _zrow(i, carry):
        zbuf[i] = jnp.zeros((DH,), jnp.float32)
        return carry

    lax.fori_loop(0, RPT_LAST, _zrow, 0)
    cp_src.wait()
    cp_dst.wait()
    cp_w.wait()

    @pl.when(sid < 15)
    def _():
        pltpu.sync_copy(zbuf.at[pl.ds(0, RPT)], acc.at[pl.ds(sid * RPT, RPT)])

    @pl.when(sid == 15)
    def _():
        pltpu.sync_copy(zbuf, acc.at[pl.ds(15 * RPT, RPT_LAST)])

    plsc.subcore_barrier()

    for k in range(NBUF):
        pltpu.async_copy(table.at[src_v.at[k]], g_rows.at[k], g_sem.at[k])

    def _group(g, carry):
        for k in range(NBUF):
            b = g * NBUF + k
            pltpu.make_async_copy(table.at[src_v.at[b]], g_rows.at[k],
                                  g_sem.at[k]).wait()

            @pl.when(g > 0)
            def _():
                pltpu.make_async_copy(s_rows.at[k], acc.at[dst_v.at[b - NBUF]],
                                      s_sem.at[k]).wait()

            for gg in range(CB // 16):
                wv = w_v[b, pl.ds(gg * 16, 16)]
                for l in range(16):
                    i = gg * 16 + l
                    s_rows[k, i] = g_rows[k, i] * wv[l]

            @pl.when(g < NB // NBUF - 1)
            def _():
                pltpu.async_copy(table.at[src_v.at[b + NBUF]], g_rows.at[k],
                                 g_sem.at[k])

            pltpu.async_copy(s_rows.at[k], acc.at[dst_v.at[b]], s_sem.at[k],
                             add=True)
        return carry

    lax.fori_loop(0, NB // NBUF, _group, 0)
    for k in range(NBUF):
        pltpu.make_async_copy(s_rows.at[k], acc.at[dst_v.at[NB - NBUF + k]],
                              s_sem.at[k]).wait()

    plsc.subcore_barrier()

    @pl.when(sid < 15)
    def _():
        pltpu.sync_copy(acc.at[pl.ds(sid * RPT, RPT)],
                        out.at[cid, pl.ds(sid * RPT, RPT)])

    @pl.when(sid == 15)
    def _():
        pltpu.sync_copy(acc.at[pl.ds(15 * RPT, RPT_LAST)],
                        out.at[cid, pl.ds(15 * RPT, RPT_LAST)])


def _lin1_body(x_ref, w_ref, xr_ref):
    xr_ref[...] = jnp.dot(x_ref[...], w_ref[...],
                          preferred_element_type=jnp.float32)


def _lin1(x, w_rel1):
    return pl.pallas_call(
        _lin1_body,
        in_specs=[pl.BlockSpec((N, D_IN), lambda: (0, 0)),
                  pl.BlockSpec((D_IN, DH), lambda: (0, 0))],
        out_specs=pl.BlockSpec((N, DH), lambda: (0, 0)),
        out_shape=jax.ShapeDtypeStruct((N, DH), jnp.float32),
    )(x, w_rel1)


def _hidden_body(p_ref, x_ref, w_ref, b_ref, h_ref):
    xo = jnp.dot(x_ref[...], w_ref[...], preferred_element_type=jnp.float32)
    h_ref[...] = jnp.maximum(p_ref[0] + p_ref[1] + xo + b_ref[...], 0.0)


def _hidden(p, x, w_root1, b1):
    return pl.pallas_call(
        _hidden_body,
        in_specs=[pl.BlockSpec((2, N, DH), lambda: (0, 0, 0)),
                  pl.BlockSpec((N, D_IN), lambda: (0, 0)),
                  pl.BlockSpec((D_IN, DH), lambda: (0, 0)),
                  pl.BlockSpec((1, DH), lambda: (0, 0))],
        out_specs=pl.BlockSpec((N, DH), lambda: (0, 0)),
        out_shape=jax.ShapeDtypeStruct((N, DH), jnp.float32),
    )(p, x, w_root1, b1)


def _out_body(q_ref, h_ref, wr_ref, wo_ref, b_ref, o_ref):
    agg = q_ref[0] + q_ref[1]
    o_ref[...] = (jnp.dot(agg, wr_ref[...], preferred_element_type=jnp.float32)
                  + jnp.dot(h_ref[...], wo_ref[...],
                            preferred_element_type=jnp.float32)
                  + b_ref[...])


def _out(q, h, wr, wo, b2):
    return pl.pallas_call(
        _out_body,
        in_specs=[pl.BlockSpec((2, N, DH), lambda: (0, 0, 0)),
                  pl.BlockSpec((N, DH), lambda: (0, 0)),
                  pl.BlockSpec((DH, DOUT), lambda: (0, 0)),
                  pl.BlockSpec((DH, DOUT), lambda: (0, 0)),
                  pl.BlockSpec((1, DOUT), lambda: (0, 0))],
        out_specs=pl.BlockSpec((N, DOUT), lambda: (0, 0)),
        out_shape=jax.ShapeDtypeStruct((N, DOUT), jnp.float32),
    )(q, h, wr, wo, b2)


def kernel(x, edge_index, edge_attr, W_rel1, b_rel1, W_root1, W_rel2, b_rel2,
           W_root2):
    e = edge_attr.shape[0]
    pad = E_PAD - e
    src3 = jnp.pad(edge_index[0], (0, pad)).reshape(NW, NB, CB)
    dst3 = jnp.pad(edge_index[1], (0, pad)).reshape(NW, NB, CB)
    w3 = jnp.pad(edge_attr, (0, pad)).reshape(NW, NB, CB)

    xr = _lin1(x, W_rel1)
    p = _sc_edge_agg(xr, src3, dst3, w3)
    h = _hidden(p, x, W_root1, b_rel1.reshape(1, DH))
    q = _sc_edge_agg(h, src3, dst3, w3)
    return _out(q, h, W_rel2, W_root2, b_rel2.reshape(1, DOUT))
